# baseline (speedup 1.0000x reference)
import math

import jax
import jax.numpy as jnp
from jax import lax
from jax.experimental import pallas as pl
from jax.experimental.pallas import tpu as pltpu
from jax.experimental.pallas import tpu_sc as plsc

_N = 10000
_E = 320000
_DH = 128
_T1 = 128
_T2 = 16
_THRESH = 1.0 / (_T1 - 1)

_CH = 80
_NW = 32
_CPT = 128
_NBUF = 4
_NSTG = _CPT // 8
_EPT = _CPT * _CH
_REAL_EPT = _E // _NW
_E_PAD = _EPT * _NW
_RPT = 632
_N_PAD = _RPT * 16

_R = 1000
_G = _N // _R


def _mm(a, b):
    return jnp.dot(a, b, preferred_element_type=jnp.float32)


def _mmT(a, b):
    return lax.dot_general(a, b, (((0,), (0,)), ((), ())),
                           preferred_element_type=jnp.float32)


def _sc_segsum_body(src_hbm, gi2_hbm, si2_hbm, z_hbm, out_hbm, *scr):
    gi_r, si_r = scr[0], scr[1]
    bufs = list(scr[2:2 + _NBUF])
    gsems = list(scr[2 + _NBUF:2 + 2 * _NBUF])
    ssems = list(scr[2 + 2 * _NBUF:2 + 3 * _NBUF])
    isems = list(scr[2 + 3 * _NBUF:4 + 3 * _NBUF])
    acc_sh = scr[4 + 3 * _NBUF]
    cid = lax.axis_index("c")
    sid = lax.axis_index("s")
    wid = sid * 2 + cid
    pltpu.sync_copy(z_hbm, acc_sh.at[pl.ds(sid * _RPT, _RPT)])
    plsc.subcore_barrier()

    def _stage(gg, p):
        off = wid * _CPT + gg * 8
        pltpu.async_copy(gi2_hbm.at[pl.ds(off, 8)], gi_r.at[p], isems[p])
        pltpu.async_copy(si2_hbm.at[pl.ds(off, 8)], si_r.at[p], isems[p])

    def _wait_stage(gg, p):
        off = wid * _CPT + gg * 8
        pltpu.make_async_copy(gi2_hbm.at[pl.ds(off, 8)], gi_r.at[p],
                              isems[p]).wait()
        pltpu.make_async_copy(si2_hbm.at[pl.ds(off, 8)], si_r.at[p],
                              isems[p]).wait()

    def _gather(b, p, r):
        pltpu.async_copy(src_hbm.at[gi_r.at[p, r]], bufs[b], gsems[b],
                         priority=1)

    def _wait_gather(b, p, r):
        pltpu.make_async_copy(src_hbm.at[gi_r.at[p, r]], bufs[b],
                              gsems[b]).wait()

    def _scatter(b, p, r):
        pltpu.async_copy(bufs[b], acc_sh.at[si_r.at[p, r]], ssems[b],
                        add=True)

    def _wait_scatter(b, p, r):
        pltpu.make_async_copy(bufs[b], acc_sh.at[si_r.at[p, r]],
                              ssems[b]).wait()

    def _group(pg, h, pn, hn, pre_b=None, issue_next=True):
        for b in range(_NBUF):
            _wait_gather(b, pg, h * _NBUF + b)
            _scatter(b, pg, h * _NBUF + b)
        if pre_b is not None:
            pre_b()
        for b in range(_NBUF):
            _wait_scatter(b, pg, h * _NBUF + b)
            if issue_next:
                _gather(b, pn, hn * _NBUF + b)

    _stage(0, 0)
    _stage(1, 1)
    _wait_stage(0, 0)
    for b in range(_NBUF):
        _gather(b, 0, b)

    def superbody(u, c):
        gg0 = 2 * u
        _group(0, 0, 0, 1)
        _group(0, 1, 1, 0, pre_b=lambda: _wait_stage(gg0 + 1, 1))
        _stage(gg0 + 2, 0)
        _group(1, 0, 1, 1)
        _group(1, 1, 0, 0, pre_b=lambda: _wait_stage(gg0 + 2, 0))
        _stage(gg0 + 3, 1)
        return c

    lax.fori_loop(0, _NSTG // 2 - 1, superbody, 0)
    _group(0, 0, 0, 1)
    _group(0, 1, 1, 0, pre_b=lambda: _wait_stage(_NSTG - 1, 1))
    _group(1, 0, 1, 1)
    _group(1, 1, 0, 0, issue_next=False)

    plsc.subcore_barrier()
    pltpu.sync_copy(acc_sh.at[pl.ds(sid * _RPT, _RPT)],
                    out_hbm.at[cid, pl.ds(sid * _RPT, _RPT)])


_sc_cache = {}


def _get_sc_segsum():
    if "segsum" not in _sc_cache:
        mesh = plsc.VectorSubcoreMesh(core_axis_name="c", subcore_axis_name="s",
                                      num_cores=2, num_subcores=16)
        _sc_cache["segsum"] = pl.kernel(
            _sc_segsum_body,
            out_type=jax.ShapeDtypeStruct((2, _N_PAD, _DH), jnp.float32),
            mesh=mesh,
            scratch_types=(
                [pltpu.VMEM((2, 8, _CH), jnp.int32),
                 pltpu.VMEM((2, 8, _CH), jnp.int32)]
                + [pltpu.VMEM((_CH, _DH), jnp.float32)] * _NBUF
                + [pltpu.SemaphoreType.DMA] * (2 * _NBUF + 2)
                + [pltpu.VMEM_SHARED((_N_PAD, _DH), jnp.float32)]))
    return _sc_cache["segsum"]


def _sc_pass1(x, row_g2, col_s2, z_init):
    return _get_sc_segsum()(x, row_g2, col_s2, z_init)


def _sc_pass2(s1s, col_g2, row_s2, z_init):
    return _get_sc_segsum()(s1s, col_g2, row_s2, z_init)


def _tc1_body(x_ref, ap_ref, wr_ref, br_ref, wo_ref, wp_ref, bp_ref, g_ref,
              b_ref, s1_ref, s1s_ref, px_ref, sts_ref):
    i = pl.program_id(0)
    agg = ap_ref[0] + ap_ref[1]
    x1 = jnp.maximum(
        _mm(agg, wr_ref[...]) + br_ref[...] + _mm(x_ref[...], wo_ref[...]),
        0.0)
    z = _mm(x1, wp_ref[...]) + bp_ref[...]
    m = jnp.mean(z, axis=-1, keepdims=True)
    v = jnp.mean((z - m) ** 2, axis=-1, keepdims=True)
    s1 = (z - m) / jnp.sqrt(v + 1e-5) * g_ref[...] + b_ref[...]
    s1_ref[...] = s1
    mx = jnp.max(s1, axis=-1, keepdims=True)
    ex = jnp.exp(s1 - mx)
    s1s = ex / jnp.sum(ex, axis=-1, keepdims=True)
    s1s_ref[...] = s1s

    @pl.when(i == 0)
    def _():
        px_ref[...] = jnp.zeros_like(px_ref)
        sts_ref[...] = jnp.zeros_like(sts_ref)

    px_ref[...] += _mmT(s1s, x1)
    sts_ref[...] += _mmT(s1s, s1s)


_tc1 = pl.pallas_call(
    _tc1_body,
    grid=(_G,),
    in_specs=[
        pl.BlockSpec((_R, _DH), lambda i: (i, 0)),
        pl.BlockSpec((2, _R, _DH), lambda i: (0, i, 0)),
        pl.BlockSpec((_DH, _DH), lambda i: (0, 0)),
        pl.BlockSpec((1, _DH), lambda i: (0, 0)),
        pl.BlockSpec((_DH, _DH), lambda i: (0, 0)),
        pl.BlockSpec((_DH, _T1), lambda i: (0, 0)),
        pl.BlockSpec((1, _T1), lambda i: (0, 0)),
        pl.BlockSpec((1, _T1), lambda i: (0, 0)),
        pl.BlockSpec((1, _T1), lambda i: (0, 0)),
    ],
    out_specs=[
        pl.BlockSpec((_R, _T1), lambda i: (i, 0)),
        pl.BlockSpec((_R, _T1), lambda i: (i, 0)),
        pl.BlockSpec((_T1, _DH), lambda i: (0, 0)),
        pl.BlockSpec((_T1, _T1), lambda i: (0, 0)),
    ],
    out_shape=[
        jax.ShapeDtypeStruct((_N, _T1), jnp.float32),
        jax.ShapeDtypeStruct((_N, _T1), jnp.float32),
        jax.ShapeDtypeStruct((_T1, _DH), jnp.float32),
        jax.ShapeDtypeStruct((_T1, _T1), jnp.float32),
    ],
)


def _tc2_body(ap_ref, s1s_ref, px_ref, sts_ref, wr2_ref, br2_ref,
              wo2_ref, wp2_ref, bp2_ref, g2_ref, b2_ref,
              mc1_ref, o1_ref, mc2_ref, o2_ref, ab_ref, s2_ref, an2_ref,
              pacc_ref, den_ref):
    i = pl.program_id(0)

    @pl.when(i == 0)
    def _():
        pacc_ref[...] = jnp.zeros_like(pacc_ref)
        den_ref[...] = jnp.zeros_like(den_ref)

    asb = ap_ref[0] + ap_ref[1]
    s1s_b = s1s_ref[...]
    pacc_ref[...] += _mmT(s1s_b, asb)
    deg_b = jnp.sum(asb, axis=1, keepdims=True)
    q_b = jnp.sum(s1s_b * s1s_b, axis=1, keepdims=True)
    den_ref[...] += jnp.reshape(jnp.sum(deg_b * q_b), (1, 1))

    @pl.when(i == _G - 1)
    def _():
        P = pacc_ref[...]
        eye1 = (lax.broadcasted_iota(jnp.int32, (_T1, _T1), 0)
                == lax.broadcasted_iota(jnp.int32, (_T1, _T1), 1)
                ).astype(jnp.float32)
        num1 = jnp.sum(P * eye1)
        den1 = den_ref[0, 0] + 1e-10
        mc1 = -num1 / den1
        sts = sts_ref[...]
        nrm = jnp.sqrt(jnp.sum(sts * sts))
        o1m = sts / (nrm + 1e-10) - eye1 / math.sqrt(_T1)
        o1 = jnp.sqrt(jnp.sum(o1m * o1m))
        adj_sq = P * (1.0 - eye1)
        d1 = jnp.sqrt(jnp.sum(adj_sq, axis=1, keepdims=True))
        dinv = 1.0 / (d1 + 1e-15)
        adj_norm1 = adj_sq * dinv * dinv.reshape(1, _T1)
        adj_bin = jnp.where(adj_norm1 > _THRESH, 1.0, 0.0).astype(jnp.float32)
        ab_ref[...] = adj_bin

        px = px_ref[...]
        agg2 = _mmT(adj_bin, px)
        x2 = jnp.maximum(
            _mm(agg2, wr2_ref[...]) + br2_ref[...] + _mm(px, wo2_ref[...]),
            0.0)
        z2 = _mm(x2, wp2_ref[...]) + bp2_ref[...]
        m2 = jnp.mean(z2, axis=-1, keepdims=True)
        v2 = jnp.mean((z2 - m2) ** 2, axis=-1, keepdims=True)
        s2 = (z2 - m2) / jnp.sqrt(v2 + 1e-5) * g2_ref[...] + b2_ref[...]
        s2_ref[...] = s2
        mx2 = jnp.max(s2, axis=-1, keepdims=True)
        ex2 = jnp.exp(s2 - mx2)
        s2s = ex2 / jnp.sum(ex2, axis=-1, keepdims=True)
        a_s2 = _mm(adj_bin, s2s)
        P2 = _mmT(s2s, a_s2)
        eye2 = (lax.broadcasted_iota(jnp.int32, (_T2, _T2), 0)
                == lax.broadcasted_iota(jnp.int32, (_T2, _T2), 1)
                ).astype(jnp.float32)
        num2 = jnp.sum(P2 * eye2)
        deg2 = jnp.sum(adj_bin, axis=1, keepdims=True)
        den2 = jnp.sum(deg2 * jnp.sum(s2s * s2s, axis=1, keepdims=True)) + 1e-10
        mc2 = -num2 / den2
        sts2 = _mmT(s2s, s2s)
        nrm2 = jnp.sqrt(jnp.sum(sts2 * sts2))
        o2m = sts2 / (nrm2 + 1e-10) - eye2 / math.sqrt(_T2)
        o2 = jnp.sqrt(jnp.sum(o2m * o2m))
        adj_sq2 = P2 * (1.0 - eye2)
        d2 = jnp.sqrt(jnp.sum(adj_sq2, axis=1, keepdims=True))
        dinv2 = 1.0 / (d2 + 1e-15)
        an2_ref[...] = adj_sq2 * dinv2 * dinv2.reshape(1, _T2)

        mc1_ref[...] = jnp.reshape(mc1, (1, 1))
        o1_ref[...] = jnp.reshape(o1, (1, 1))
        mc2_ref[...] = jnp.reshape(mc2, (1, 1))
        o2_ref[...] = jnp.reshape(o2, (1, 1))


_tc2 = pl.pallas_call(
    _tc2_body,
    grid=(_G,),
    in_specs=[
        pl.BlockSpec((2, _R, _T1), lambda i: (0, i, 0)),
        pl.BlockSpec((_R, _T1), lambda i: (i, 0)),
        pl.BlockSpec((_T1, _DH), lambda i: (0, 0)),
        pl.BlockSpec((_T1, _T1), lambda i: (0, 0)),
        pl.BlockSpec((_DH, _DH), lambda i: (0, 0)),
        pl.BlockSpec((1, _DH), lambda i: (0, 0)),
        pl.BlockSpec((_DH, _DH), lambda i: (0, 0)),
        pl.BlockSpec((_DH, _T2), lambda i: (0, 0)),
        pl.BlockSpec((1, _T2), lambda i: (0, 0)),
        pl.BlockSpec((1, _T2), lambda i: (0, 0)),
        pl.BlockSpec((1, _T2), lambda i: (0, 0)),
    ],
    out_specs=[
        pl.BlockSpec((1, 1), lambda i: (0, 0)),
        pl.BlockSpec((1, 1), lambda i: (0, 0)),
        pl.BlockSpec((1, 1), lambda i: (0, 0)),
        pl.BlockSpec((1, 1), lambda i: (0, 0)),
        pl.BlockSpec((_T1, _T1), lambda i: (0, 0)),
        pl.BlockSpec((_T1, _T2), lambda i: (0, 0)),
        pl.BlockSpec((_T2, _T2), lambda i: (0, 0)),
    ],
    out_shape=[
        jax.ShapeDtypeStruct((1, 1), jnp.float32),
        jax.ShapeDtypeStruct((1, 1), jnp.float32),
        jax.ShapeDtypeStruct((1, 1), jnp.float32),
        jax.ShapeDtypeStruct((1, 1), jnp.float32),
        jax.ShapeDtypeStruct((_T1, _T1), jnp.float32),
        jax.ShapeDtypeStruct((_T1, _T2), jnp.float32),
        jax.ShapeDtypeStruct((_T2, _T2), jnp.float32),
    ],
    scratch_shapes=[pltpu.VMEM((_T1, _T1), jnp.float32),
                    pltpu.VMEM((1, 1), jnp.float32)],
)


def kernel(x, edge_index, W_rel1, b_rel1, W_root1, W_pool1, b_pool1, ln1_g,
           ln1_b, W_rel2, b_rel2, W_root2, W_pool2, b_pool2, ln2_g, ln2_b):
    row = edge_index[0]
    col = edge_index[1]
    pad = _EPT - _REAL_EPT

    def _padded(idx, fill):
        return jnp.concatenate(
            [idx.reshape(_NW, _REAL_EPT),
             jnp.full((_NW, pad), fill, jnp.int32)], axis=1)

    row_g2 = _padded(row, 0).reshape(_E_PAD // _CH, _CH)
    col_s2 = _padded(col, _N).reshape(_E_PAD // _CH, _CH)
    col_g2 = _padded(col, 0).reshape(_E_PAD // _CH, _CH)
    row_s2 = _padded(row, _N).reshape(_E_PAD // _CH, _CH)
    z_init = jnp.zeros((_RPT, _DH), jnp.float32)

    agg_parts = _sc_pass1(x, row_g2, col_s2, z_init)
    s1, s1s, pooled_x1, sts1 = _tc1(
        x, agg_parts, W_rel1, b_rel1.reshape(1, -1), W_root1, W_pool1,
        b_pool1.reshape(1, -1), ln1_g.reshape(1, -1), ln1_b.reshape(1, -1))
    as_parts = _sc_pass2(s1s, col_g2, row_s2, z_init)
    mc1, o1, mc2, o2, adj_bin, s2, adj_norm2 = _tc2(
        as_parts, s1s, pooled_x1, sts1, W_rel2,
        b_rel2.reshape(1, -1), W_root2, W_pool2, b_pool2.reshape(1, -1),
        ln2_g.reshape(1, -1), ln2_b.reshape(1, -1))
    return (mc1.reshape(()), o1.reshape(()), mc2.reshape(()), o2.reshape(()),
            s1, adj_bin, s2, adj_norm2)


# speedup vs baseline: 1.0011x; 1.0011x over previous
import math

import jax
import jax.numpy as jnp
from jax import lax
from jax.experimental import pallas as pl
from jax.experimental.pallas import tpu as pltpu
from jax.experimental.pallas import tpu_sc as plsc

_N = 10000
_E = 320000
_DH = 128
_T1 = 128
_T2 = 16
_THRESH = 1.0 / (_T1 - 1)

_CH = 80
_NW = 32
_CPT = 128
_NBUF = 4
_NSTG = _CPT // 8
_EPT = _CPT * _CH
_REAL_EPT = _E // _NW
_E_PAD = _EPT * _NW
_RPT = 632
_N_PAD = _RPT * 16

_R = 1000
_G = _N // _R


def _mm(a, b):
    return jnp.dot(a, b, preferred_element_type=jnp.float32)


def _mmT(a, b):
    return lax.dot_general(a, b, (((0,), (0,)), ((), ())),
                           preferred_element_type=jnp.float32)


def _sc_segsum_body(src_hbm, gi2_hbm, si2_hbm, z_hbm, out_hbm, *scr):
    gi_r, si_r = scr[0], scr[1]
    bufs = list(scr[2:2 + _NBUF])
    gsems = list(scr[2 + _NBUF:2 + 2 * _NBUF])
    ssems = list(scr[2 + 2 * _NBUF:2 + 3 * _NBUF])
    isems = list(scr[2 + 3 * _NBUF:4 + 3 * _NBUF])
    acc_sh = scr[4 + 3 * _NBUF]
    cid = lax.axis_index("c")
    sid = lax.axis_index("s")
    wid = sid * 2 + cid
    pltpu.sync_copy(z_hbm, acc_sh.at[pl.ds(sid * _RPT, _RPT)])
    plsc.subcore_barrier()

    def _stage(gg, p):
        off = wid * _CPT + gg * 8
        pltpu.async_copy(gi2_hbm.at[pl.ds(off, 8)], gi_r.at[p], isems[p])
        pltpu.async_copy(si2_hbm.at[pl.ds(off, 8)], si_r.at[p], isems[p])

    def _wait_stage(gg, p):
        off = wid * _CPT + gg * 8
        pltpu.make_async_copy(gi2_hbm.at[pl.ds(off, 8)], gi_r.at[p],
                              isems[p]).wait()
        pltpu.make_async_copy(si2_hbm.at[pl.ds(off, 8)], si_r.at[p],
                              isems[p]).wait()

    def _gather(b, p, r):
        pltpu.async_copy(src_hbm.at[gi_r.at[p, r]], bufs[b], gsems[b])

    def _wait_gather(b, p, r):
        pltpu.make_async_copy(src_hbm.at[gi_r.at[p, r]], bufs[b],
                              gsems[b]).wait()

    def _scatter(b, p, r):
        pltpu.async_copy(bufs[b], acc_sh.at[si_r.at[p, r]], ssems[b],
                        add=True)

    def _wait_scatter(b, p, r):
        pltpu.make_async_copy(bufs[b], acc_sh.at[si_r.at[p, r]],
                              ssems[b]).wait()

    def _group(pg, h, pn, hn, pre_b=None, issue_next=True):
        for b in range(_NBUF):
            _wait_gather(b, pg, h * _NBUF + b)
            _scatter(b, pg, h * _NBUF + b)
        if pre_b is not None:
            pre_b()
        for b in range(_NBUF):
            _wait_scatter(b, pg, h * _NBUF + b)
            if issue_next:
                _gather(b, pn, hn * _NBUF + b)

    _stage(0, 0)
    _stage(1, 1)
    _wait_stage(0, 0)
    for b in range(_NBUF):
        _gather(b, 0, b)

    def superbody(u, c):
        gg0 = 2 * u
        _group(0, 0, 0, 1)
        _group(0, 1, 1, 0, pre_b=lambda: _wait_stage(gg0 + 1, 1))
        _stage(gg0 + 2, 0)
        _group(1, 0, 1, 1)
        _group(1, 1, 0, 0, pre_b=lambda: _wait_stage(gg0 + 2, 0))
        _stage(gg0 + 3, 1)
        return c

    lax.fori_loop(0, _NSTG // 2 - 1, superbody, 0)
    _group(0, 0, 0, 1)
    _group(0, 1, 1, 0, pre_b=lambda: _wait_stage(_NSTG - 1, 1))
    _group(1, 0, 1, 1)
    _group(1, 1, 0, 0, issue_next=False)

    plsc.subcore_barrier()
    pltpu.sync_copy(acc_sh.at[pl.ds(sid * _RPT, _RPT)],
                    out_hbm.at[cid, pl.ds(sid * _RPT, _RPT)])


_sc_cache = {}


def _get_sc_segsum():
    if "segsum" not in _sc_cache:
        mesh = plsc.VectorSubcoreMesh(core_axis_name="c", subcore_axis_name="s",
                                      num_cores=2, num_subcores=16)
        _sc_cache["segsum"] = pl.kernel(
            _sc_segsum_body,
            out_type=jax.ShapeDtypeStruct((2, _N_PAD, _DH), jnp.float32),
            mesh=mesh,
            scratch_types=(
                [pltpu.VMEM((2, 8, _CH), jnp.int32),
                 pltpu.VMEM((2, 8, _CH), jnp.int32)]
                + [pltpu.VMEM((_CH, _DH), jnp.float32)] * _NBUF
                + [pltpu.SemaphoreType.DMA] * (2 * _NBUF + 2)
                + [pltpu.VMEM_SHARED((_N_PAD, _DH), jnp.float32)]))
    return _sc_cache["segsum"]


def _sc_pass1(x, row_g2, col_s2, z_init):
    return _get_sc_segsum()(x, row_g2, col_s2, z_init)


def _sc_pass2(s1s, col_g2, row_s2, z_init):
    return _get_sc_segsum()(s1s, col_g2, row_s2, z_init)


def _tc1_body(x_ref, ap_ref, wr_ref, br_ref, wo_ref, wp_ref, bp_ref, g_ref,
              b_ref, s1_ref, s1s_ref, px_ref, sts_ref):
    i = pl.program_id(0)
    agg = ap_ref[0] + ap_ref[1]
    x1 = jnp.maximum(
        _mm(agg, wr_ref[...]) + br_ref[...] + _mm(x_ref[...], wo_ref[...]),
        0.0)
    z = _mm(x1, wp_ref[...]) + bp_ref[...]
    m = jnp.mean(z, axis=-1, keepdims=True)
    v = jnp.mean((z - m) ** 2, axis=-1, keepdims=True)
    s1 = (z - m) / jnp.sqrt(v + 1e-5) * g_ref[...] + b_ref[...]
    s1_ref[...] = s1
    mx = jnp.max(s1, axis=-1, keepdims=True)
    ex = jnp.exp(s1 - mx)
    s1s = ex / jnp.sum(ex, axis=-1, keepdims=True)
    s1s_ref[...] = s1s

    @pl.when(i == 0)
    def _():
        px_ref[...] = jnp.zeros_like(px_ref)
        sts_ref[...] = jnp.zeros_like(sts_ref)

    px_ref[...] += _mmT(s1s, x1)
    sts_ref[...] += _mmT(s1s, s1s)


_tc1 = pl.pallas_call(
    _tc1_body,
    grid=(_G,),
    in_specs=[
        pl.BlockSpec((_R, _DH), lambda i: (i, 0)),
        pl.BlockSpec((2, _R, _DH), lambda i: (0, i, 0)),
        pl.BlockSpec((_DH, _DH), lambda i: (0, 0)),
        pl.BlockSpec((1, _DH), lambda i: (0, 0)),
        pl.BlockSpec((_DH, _DH), lambda i: (0, 0)),
        pl.BlockSpec((_DH, _T1), lambda i: (0, 0)),
        pl.BlockSpec((1, _T1), lambda i: (0, 0)),
        pl.BlockSpec((1, _T1), lambda i: (0, 0)),
        pl.BlockSpec((1, _T1), lambda i: (0, 0)),
    ],
    out_specs=[
        pl.BlockSpec((_R, _T1), lambda i: (i, 0)),
        pl.BlockSpec((_R, _T1), lambda i: (i, 0)),
        pl.BlockSpec((_T1, _DH), lambda i: (0, 0)),
        pl.BlockSpec((_T1, _T1), lambda i: (0, 0)),
    ],
    out_shape=[
        jax.ShapeDtypeStruct((_N, _T1), jnp.float32),
        jax.ShapeDtypeStruct((_N, _T1), jnp.float32),
        jax.ShapeDtypeStruct((_T1, _DH), jnp.float32),
        jax.ShapeDtypeStruct((_T1, _T1), jnp.float32),
    ],
)


def _tc2_body(ap_ref, s1s_ref, px_ref, sts_ref, wr2_ref, br2_ref,
              wo2_ref, wp2_ref, bp2_ref, g2_ref, b2_ref,
              mc1_ref, o1_ref, mc2_ref, o2_ref, ab_ref, s2_ref, an2_ref,
              pacc_ref, den_ref):
    i = pl.program_id(0)

    @pl.when(i == 0)
    def _():
        pacc_ref[...] = jnp.zeros_like(pacc_ref)
        den_ref[...] = jnp.zeros_like(den_ref)

    asb = ap_ref[0] + ap_ref[1]
    s1s_b = s1s_ref[...]
    pacc_ref[...] += _mmT(s1s_b, asb)
    deg_b = jnp.sum(asb, axis=1, keepdims=True)
    q_b = jnp.sum(s1s_b * s1s_b, axis=1, keepdims=True)
    den_ref[...] += jnp.reshape(jnp.sum(deg_b * q_b), (1, 1))

    @pl.when(i == _G - 1)
    def _():
        P = pacc_ref[...]
        eye1 = (lax.broadcasted_iota(jnp.int32, (_T1, _T1), 0)
                == lax.broadcasted_iota(jnp.int32, (_T1, _T1), 1)
                ).astype(jnp.float32)
        num1 = jnp.sum(P * eye1)
        den1 = den_ref[0, 0] + 1e-10
        mc1 = -num1 / den1
        sts = sts_ref[...]
        nrm = jnp.sqrt(jnp.sum(sts * sts))
        o1m = sts / (nrm + 1e-10) - eye1 / math.sqrt(_T1)
        o1 = jnp.sqrt(jnp.sum(o1m * o1m))
        adj_sq = P * (1.0 - eye1)
        d1 = jnp.sqrt(jnp.sum(adj_sq, axis=1, keepdims=True))
        dinv = 1.0 / (d1 + 1e-15)
        adj_norm1 = adj_sq * dinv * dinv.reshape(1, _T1)
        adj_bin = jnp.where(adj_norm1 > _THRESH, 1.0, 0.0).astype(jnp.float32)
        ab_ref[...] = adj_bin

        px = px_ref[...]
        agg2 = _mmT(adj_bin, px)
        x2 = jnp.maximum(
            _mm(agg2, wr2_ref[...]) + br2_ref[...] + _mm(px, wo2_ref[...]),
            0.0)
        z2 = _mm(x2, wp2_ref[...]) + bp2_ref[...]
        m2 = jnp.mean(z2, axis=-1, keepdims=True)
        v2 = jnp.mean((z2 - m2) ** 2, axis=-1, keepdims=True)
        s2 = (z2 - m2) / jnp.sqrt(v2 + 1e-5) * g2_ref[...] + b2_ref[...]
        s2_ref[...] = s2
        mx2 = jnp.max(s2, axis=-1, keepdims=True)
        ex2 = jnp.exp(s2 - mx2)
        s2s = ex2 / jnp.sum(ex2, axis=-1, keepdims=True)
        a_s2 = _mm(adj_bin, s2s)
        P2 = _mmT(s2s, a_s2)
        eye2 = (lax.broadcasted_iota(jnp.int32, (_T2, _T2), 0)
                == lax.broadcasted_iota(jnp.int32, (_T2, _T2), 1)
                ).astype(jnp.float32)
        num2 = jnp.sum(P2 * eye2)
        deg2 = jnp.sum(adj_bin, axis=1, keepdims=True)
        den2 = jnp.sum(deg2 * jnp.sum(s2s * s2s, axis=1, keepdims=True)) + 1e-10
        mc2 = -num2 / den2
        sts2 = _mmT(s2s, s2s)
        nrm2 = jnp.sqrt(jnp.sum(sts2 * sts2))
        o2m = sts2 / (nrm2 + 1e-10) - eye2 / math.sqrt(_T2)
        o2 = jnp.sqrt(jnp.sum(o2m * o2m))
        adj_sq2 = P2 * (1.0 - eye2)
        d2 = jnp.sqrt(jnp.sum(adj_sq2, axis=1, keepdims=True))
        dinv2 = 1.0 / (d2 + 1e-15)
        an2_ref[...] = adj_sq2 * dinv2 * dinv2.reshape(1, _T2)

        mc1_ref[...] = jnp.reshape(mc1, (1, 1))
        o1_ref[...] = jnp.reshape(o1, (1, 1))
        mc2_ref[...] = jnp.reshape(mc2, (1, 1))
        o2_ref[...] = jnp.reshape(o2, (1, 1))


_tc2 = pl.pallas_call(
    _tc2_body,
    grid=(_G,),
    in_specs=[
        pl.BlockSpec((2, _R, _T1), lambda i: (0, i, 0)),
        pl.BlockSpec((_R, _T1), lambda i: (i, 0)),
        pl.BlockSpec((_T1, _DH), lambda i: (0, 0)),
        pl.BlockSpec((_T1, _T1), lambda i: (0, 0)),
        pl.BlockSpec((_DH, _DH), lambda i: (0, 0)),
        pl.BlockSpec((1, _DH), lambda i: (0, 0)),
        pl.BlockSpec((_DH, _DH), lambda i: (0, 0)),
        pl.BlockSpec((_DH, _T2), lambda i: (0, 0)),
        pl.BlockSpec((1, _T2), lambda i: (0, 0)),
        pl.BlockSpec((1, _T2), lambda i: (0, 0)),
        pl.BlockSpec((1, _T2), lambda i: (0, 0)),
    ],
    out_specs=[
        pl.BlockSpec((1, 1), lambda i: (0, 0)),
        pl.BlockSpec((1, 1), lambda i: (0, 0)),
        pl.BlockSpec((1, 1), lambda i: (0, 0)),
        pl.BlockSpec((1, 1), lambda i: (0, 0)),
        pl.BlockSpec((_T1, _T1), lambda i: (0, 0)),
        pl.BlockSpec((_T1, _T2), lambda i: (0, 0)),
        pl.BlockSpec((_T2, _T2), lambda i: (0, 0)),
    ],
    out_shape=[
        jax.ShapeDtypeStruct((1, 1), jnp.float32),
        jax.ShapeDtypeStruct((1, 1), jnp.float32),
        jax.ShapeDtypeStruct((1, 1), jnp.float32),
        jax.ShapeDtypeStruct((1, 1), jnp.float32),
        jax.ShapeDtypeStruct((_T1, _T1), jnp.float32),
        jax.ShapeDtypeStruct((_T1, _T2), jnp.float32),
        jax.ShapeDtypeStruct((_T2, _T2), jnp.float32),
    ],
    scratch_shapes=[pltpu.VMEM((_T1, _T1), jnp.float32),
                    pltpu.VMEM((1, 1), jnp.float32)],
)


def kernel(x, edge_index, W_rel1, b_rel1, W_root1, W_pool1, b_pool1, ln1_g,
           ln1_b, W_rel2, b_rel2, W_root2, W_pool2, b_pool2, ln2_g, ln2_b):
    row = edge_index[0]
    col = edge_index[1]
    pad = _EPT - _REAL_EPT

    def _padded(idx, fill):
        return jnp.concatenate(
            [idx.reshape(_NW, _REAL_EPT),
             jnp.full((_NW, pad), fill, jnp.int32)], axis=1)

    row_g2 = _padded(row, 0).reshape(_E_PAD // _CH, _CH)
    col_s2 = _padded(col, _N).reshape(_E_PAD // _CH, _CH)
    col_g2 = _padded(col, 0).reshape(_E_PAD // _CH, _CH)
    row_s2 = _padded(row, _N).reshape(_E_PAD // _CH, _CH)
    z_init = jnp.zeros((_RPT, _DH), jnp.float32)

    agg_parts = _sc_pass1(x, row_g2, col_s2, z_init)
    s1, s1s, pooled_x1, sts1 = _tc1(
        x, agg_parts, W_rel1, b_rel1.reshape(1, -1), W_root1, W_pool1,
        b_pool1.reshape(1, -1), ln1_g.reshape(1, -1), ln1_b.reshape(1, -1))
    as_parts = _sc_pass2(s1s, col_g2, row_s2, z_init)
    mc1, o1, mc2, o2, adj_bin, s2, adj_norm2 = _tc2(
        as_parts, s1s, pooled_x1, sts1, W_rel2,
        b_rel2.reshape(1, -1), W_root2, W_pool2, b_pool2.reshape(1, -1),
        ln2_g.reshape(1, -1), ln2_b.reshape(1, -1))
    return (mc1.reshape(()), o1.reshape(()), mc2.reshape(()), o2.reshape(()),
            s1, adj_bin, s2, adj_norm2)
